# Initial kernel scaffold; baseline (speedup 1.0000x reference)
import jax
import jax.numpy as jnp
from jax.experimental import pallas as pl


def kernel(gen_embedding, domains, proj_A, proj_B, pooler_A, pooler_B, A_item_emb, B_item_emb, data_emb):
    raise NotImplementedError("write your pallas kernel here")


# speedup vs baseline: 1.0001x; 1.0001x over previous
import jax
import jax.numpy as jnp
from jax.experimental import pallas as pl

K = 20
ANUM = 100000


def _pooling(x, pooler_w):
    length_mask = jnp.where(jnp.sum(x, axis=-1) == 0, 0, 1)
    weight = x @ pooler_w.T
    weight = jnp.where(length_mask[..., None] == 1, weight, -1000000000.0)
    return jnp.sum(jax.nn.softmax(weight, axis=1) * x, axis=1)


def _recall(gen, item_emb, data_emb, k, offset):
    sim_dot = gen @ item_emb.T
    divider = jnp.linalg.norm(gen, axis=-1, keepdims=True) * jnp.linalg.norm(item_emb, axis=-1)[None, :]
    sim_score = sim_dot / divider
    _, idx = jax.lax.top_k(sim_score, k)
    sim_rank = idx + 1 + offset
    recalled_embedding = data_emb[sim_rank.reshape(-1)].reshape(sim_rank.shape[0], sim_rank.shape[1], -1)
    return sim_rank, recalled_embedding


def kernel(gen_embedding, domains, proj_A, proj_B, pooler_A, pooler_B, A_item_emb, B_item_emb, data_emb):
    A_gen = _pooling(gen_embedding @ proj_A.T, pooler_A)
    B_gen = _pooling(gen_embedding @ proj_B.T, pooler_B)
    A_recalled, A_recalled_emb = _recall(A_gen, A_item_emb, data_emb, K, 0)
    B_recalled, B_recalled_emb = _recall(B_gen, B_item_emb, data_emb, K, ANUM)
    recalled_emb = jnp.where(domains[:, None, None] == 0, B_recalled_emb, A_recalled_emb)
    recalled = jnp.where(domains[:, None] == 0, B_recalled, A_recalled)
    return recalled, recalled_emb


# speedup vs baseline: 4.3740x; 4.3738x over previous
import functools

import jax
import jax.numpy as jnp
from jax import lax
from jax.experimental import pallas as pl
from jax.experimental.pallas import tpu as pltpu
from jax.experimental.pallas import tpu_sc as plsc

K = 20
ANUM = 100000

B = 1024
L = 50
DG = 2048
DM = 128

NPAD = 100352
NBLK = NPAD // 128
_NB = 1024
_RB = 256
T = 24

_INT_MIN = -(2**31)
_BIG = 10**9


def _sortable(x):
    s = lax.bitcast_convert_type(x, jnp.int32)
    return jnp.where(s >= 0, s, s ^ 0x7FFFFFFF)


def _simdot_body(gen_ref, item_ref, ng_ref, ni_ref, out_ref, bm_ref):
    j = pl.program_id(1)
    dot = lax.dot_general(
        gen_ref[...], item_ref[...],
        dimension_numbers=(((1,), (1,)), ((), ())),
        preferred_element_type=jnp.float32,
    )
    q = dot / (ng_ref[...] * ni_ref[0:1, :])
    q3 = q.reshape(_RB, _NB // 128, 128)
    out_ref[...] = q3
    sb = lax.broadcasted_iota(jnp.int32, (_RB, _NB // 128, 128), 1)
    ln = lax.broadcasted_iota(jnp.int32, (_RB, _NB // 128, 128), 2)
    gcol = j * _NB + sb * 128 + ln
    q3m = jnp.where(gcol < ANUM, q3, jnp.float32(-3.0e38))
    bm_ref[...] = jnp.max(q3m, axis=-1).reshape(_RB, 1, 1, _NB // 128)


def _simscore(gen, item_pad, ng, ni_pad):
    return pl.pallas_call(
        _simdot_body,
        grid=(B // _RB, NPAD // _NB),
        in_specs=[
            pl.BlockSpec((_RB, DM), lambda i, j: (i, 0)),
            pl.BlockSpec((_NB, DM), lambda i, j: (j, 0)),
            pl.BlockSpec((_RB, 1), lambda i, j: (i, 0)),
            pl.BlockSpec((8, _NB), lambda i, j: (0, j)),
        ],
        out_specs=[
            pl.BlockSpec((_RB, _NB // 128, 128), lambda i, j: (i, j, 0)),
            pl.BlockSpec((_RB, 1, 1, _NB // 128), lambda i, j: (i, j, 0, 0)),
        ],
        out_shape=[
            jax.ShapeDtypeStruct((B, NBLK, 128), jnp.float32),
            jax.ShapeDtypeStruct((B, NPAD // _NB, 1, _NB // 128), jnp.float32),
        ],
    )(gen, item_pad, ng, ni_pad)


def _selblocks_body(bm_ref, tb_ref, fl_ref):
    i = pl.program_id(0)
    key = _sortable(bm_ref[...])
    cols = lax.broadcasted_iota(jnp.int32, (_RB, NBLK), 1)
    tbs = []
    for _ in range(T):
        m = jnp.max(key, axis=-1, keepdims=True)
        c = jnp.min(jnp.where(key == m, cols, _BIG), axis=-1, keepdims=True)
        tbs.append(c)
        key = jnp.where(cols == c, _INT_MIN, key)
    tb = jnp.concatenate(tbs, axis=1)
    tb_ref[...] = tb
    rows = i * _RB + lax.broadcasted_iota(jnp.int32, (_RB, T), 0)
    fl_ref[...] = rows * NBLK + tb


def _selblocks(bm):
    return pl.pallas_call(
        _selblocks_body,
        grid=(B // _RB,),
        in_specs=[pl.BlockSpec((_RB, NBLK), lambda i: (i, 0))],
        out_specs=[
            pl.BlockSpec((_RB, T), lambda i: (i, 0)),
            pl.BlockSpec((_RB, T), lambda i: (i, 0)),
        ],
        out_shape=[
            jax.ShapeDtypeStruct((B, T), jnp.int32),
            jax.ShapeDtypeStruct((B, T), jnp.int32),
        ],
    )(bm)


_NW = 32
_CHUNK3 = B * T // _NW
_IDXROWS3 = _CHUNK3 // 128


def _sc_gather_blocks(scores_flat, flat_idx2):
    mesh = plsc.VectorSubcoreMesh(core_axis_name="c", subcore_axis_name="s")

    @functools.partial(
        pl.kernel, mesh=mesh,
        out_type=jax.ShapeDtypeStruct((B * T, 128), jnp.float32),
        scratch_types=[
            pltpu.VMEM((_IDXROWS3, 128), jnp.int32),
            pltpu.VMEM((_CHUNK3, 128), jnp.float32),
            pltpu.SemaphoreType.DMA,
        ],
    )
    def k(scores_hbm, idx_hbm, out_hbm, idx_v, rows_v, sem):
        wid = lax.axis_index("s") * 2 + lax.axis_index("c")
        pltpu.sync_copy(idx_hbm.at[wid], idx_v)
        for t in range(_IDXROWS3):
            pltpu.async_copy(scores_hbm.at[idx_v.at[t]],
                             rows_v.at[pl.ds(t * 128, 128)], sem).wait()
        pltpu.sync_copy(rows_v, out_hbm.at[pl.ds(wid * _CHUNK3, _CHUNK3)])

    return k(scores_flat, flat_idx2)


def _extract_body(cand_ref, tb_ref, out_ref):
    key = _sortable(cand_ref[...])
    tb3 = tb_ref[...][:, :, None]
    lane = lax.broadcasted_iota(jnp.int32, (_RB, T, 128), 2)
    colidx = tb3 * 128 + lane
    key = jnp.where(colidx < ANUM, key, _INT_MIN)
    outs = []
    for _ in range(K):
        m2 = jnp.max(key, axis=2, keepdims=True)
        m = jnp.max(m2, axis=1, keepdims=True)
        c2 = jnp.min(jnp.where(key == m, colidx, _BIG), axis=2, keepdims=True)
        c = jnp.min(c2, axis=1, keepdims=True)
        outs.append(c[:, 0, :])
        key = jnp.where(colidx == c, _INT_MIN, key)
    out_ref[...] = jnp.concatenate(outs, axis=1)


def _extract(cand, tb):
    return pl.pallas_call(
        _extract_body,
        grid=(B // _RB,),
        in_specs=[
            pl.BlockSpec((_RB, T, 128), lambda i: (i, 0, 0)),
            pl.BlockSpec((_RB, T), lambda i: (i, 0)),
        ],
        out_specs=pl.BlockSpec((_RB, K), lambda i: (i, 0)),
        out_shape=jax.ShapeDtypeStruct((B, K), jnp.int32),
    )(cand, tb)


_CHUNK5 = B * K // _NW
_IDXROWS5 = _CHUNK5 // 128


def _sc_gather_emb(data_emb, idx2):
    mesh = plsc.VectorSubcoreMesh(core_axis_name="c", subcore_axis_name="s")

    @functools.partial(
        pl.kernel, mesh=mesh,
        out_type=jax.ShapeDtypeStruct((B * K, DM), jnp.float32),
        scratch_types=[
            pltpu.VMEM((_IDXROWS5, 128), jnp.int32),
            pltpu.VMEM((_CHUNK5, DM), jnp.float32),
            pltpu.SemaphoreType.DMA,
        ],
    )
    def k(table_hbm, idx_hbm, out_hbm, idx_v, rows_v, sem):
        wid = lax.axis_index("s") * 2 + lax.axis_index("c")
        pltpu.sync_copy(idx_hbm.at[wid], idx_v)
        for t in range(_IDXROWS5):
            pltpu.async_copy(table_hbm.at[idx_v.at[t]],
                             rows_v.at[pl.ds(t * 128, 128)], sem).wait()
        pltpu.sync_copy(rows_v, out_hbm.at[pl.ds(wid * _CHUNK5, _CHUNK5)])

    return k(data_emb, idx2)


def _pooling(x, pooler_w):
    length_mask = jnp.where(jnp.sum(x, axis=-1) == 0, 0, 1)
    weight = x @ pooler_w.T
    weight = jnp.where(length_mask[..., None] == 1, weight, -1000000000.0)
    return jnp.sum(jax.nn.softmax(weight, axis=1) * x, axis=1)


def _topk_idx(gen, item_emb):
    n = item_emb.shape[0]
    item_pad = jnp.pad(item_emb, ((0, NPAD - n), (0, 0)))
    ng = jnp.linalg.norm(gen, axis=-1, keepdims=True)
    ni = jnp.linalg.norm(item_emb, axis=-1)
    ni_pad = jnp.broadcast_to(
        jnp.pad(ni, (0, NPAD - n), constant_values=1.0)[None, :], (8, NPAD))
    scores3, bm4 = _simscore(gen, item_pad, ng, ni_pad)
    tb, fl = _selblocks(bm4.reshape(B, NBLK))
    cand = _sc_gather_blocks(scores3.reshape(B * NBLK, 128),
                             fl.reshape(_NW, _IDXROWS3, 128))
    cand3 = cand.reshape(B, T, 128)
    return _extract(cand3, tb)


def kernel(gen_embedding, domains, proj_A, proj_B, pooler_A, pooler_B, A_item_emb, B_item_emb, data_emb):
    A_gen = _pooling(gen_embedding @ proj_A.T, pooler_A)
    B_gen = _pooling(gen_embedding @ proj_B.T, pooler_B)
    idx_A = _topk_idx(A_gen, A_item_emb)
    idx_B = _topk_idx(B_gen, B_item_emb)
    recalled = jnp.where(domains[:, None] == 0, idx_B + 1 + ANUM, idx_A + 1)
    emb = _sc_gather_emb(data_emb, recalled.reshape(_NW, _IDXROWS5, 128))
    recalled_emb = emb.reshape(B, K, DM)
    return recalled, recalled_emb


# speedup vs baseline: 6.8244x; 1.5602x over previous
import functools

import jax
import jax.numpy as jnp
from jax import lax
from jax.experimental import pallas as pl
from jax.experimental.pallas import tpu as pltpu
from jax.experimental.pallas import tpu_sc as plsc

K = 20
ANUM = 100000

B = 1024
L = 50
DG = 2048
DM = 128

NPAD = 100352
NBLK = NPAD // 128
_NB = 1024
_RB = 256
T = 24

_INT_MIN = -(2**31)
_BIG = 10**9


def _sortable(x):
    s = lax.bitcast_convert_type(x, jnp.int32)
    return jnp.where(s >= 0, s, s ^ 0x7FFFFFFF)


def _simdot_body(tsel_ref, cnt0_ref, gen_ref, itemA_ref, itemB_ref, ng_ref,
                 niA_ref, niB_ref, out_ref, bm_ref):
    i = pl.program_id(0)
    j = pl.program_id(1)
    sel = tsel_ref[i]
    cnt0 = cnt0_ref[0]

    def q_of(item_ref, ni_ref):
        dot = lax.dot_general(
            gen_ref[...], item_ref[...],
            dimension_numbers=(((1,), (1,)), ((), ())),
            preferred_element_type=jnp.float32,
        )
        q = dot / (ng_ref[...] * ni_ref[0:1, :])
        return q.reshape(_RB, _NB // 128, 128)

    @pl.when(sel != 1)
    def _():
        out_ref[...] = q_of(itemB_ref, niB_ref)

    @pl.when(sel != 0)
    def _():
        qa = q_of(itemA_ref, niA_ref)
        rows = i * _RB + lax.broadcasted_iota(jnp.int32, (_RB, _NB // 128, 128), 0)
        keep_b = jnp.logical_and(rows < cnt0, sel == 2)
        out_ref[...] = jnp.where(keep_b, out_ref[...], qa)

    q3 = out_ref[...]
    sb = lax.broadcasted_iota(jnp.int32, (_RB, _NB // 128, 128), 1)
    ln = lax.broadcasted_iota(jnp.int32, (_RB, _NB // 128, 128), 2)
    gcol = j * _NB + sb * 128 + ln
    q3m = jnp.where(gcol < ANUM, q3, jnp.float32(-3.0e38))
    bm_ref[...] = jnp.max(q3m, axis=-1).reshape(_RB, 1, 1, _NB // 128)


def _simscore(gperm, itemA_pad, itemB_pad, ng, niA_pad, niB_pad, tsel, cnt0):
    grid_spec = pltpu.PrefetchScalarGridSpec(
        num_scalar_prefetch=2,
        grid=(B // _RB, NPAD // _NB),
        in_specs=[
            pl.BlockSpec((_RB, DM), lambda i, j, t, c: (i, 0)),
            pl.BlockSpec((_NB, DM), lambda i, j, t, c: (j, 0)),
            pl.BlockSpec((_NB, DM), lambda i, j, t, c: (j, 0)),
            pl.BlockSpec((_RB, 1), lambda i, j, t, c: (i, 0)),
            pl.BlockSpec((8, _NB), lambda i, j, t, c: (0, j)),
            pl.BlockSpec((8, _NB), lambda i, j, t, c: (0, j)),
        ],
        out_specs=[
            pl.BlockSpec((_RB, _NB // 128, 128), lambda i, j, t, c: (i, j, 0)),
            pl.BlockSpec((_RB, 1, 1, _NB // 128), lambda i, j, t, c: (i, j, 0, 0)),
        ],
    )
    return pl.pallas_call(
        _simdot_body,
        grid_spec=grid_spec,
        out_shape=[
            jax.ShapeDtypeStruct((B, NBLK, 128), jnp.float32),
            jax.ShapeDtypeStruct((B, NPAD // _NB, 1, _NB // 128), jnp.float32),
        ],
    )(tsel, cnt0, gperm, itemA_pad, itemB_pad, ng, niA_pad, niB_pad)


def _selblocks_body(bm_ref, tb_ref, fl_ref):
    i = pl.program_id(0)
    key = _sortable(bm_ref[...])
    cols = lax.broadcasted_iota(jnp.int32, (_RB, NBLK), 1)
    tbs = []
    for _ in range(T):
        m = jnp.max(key, axis=-1, keepdims=True)
        c = jnp.min(jnp.where(key == m, cols, _BIG), axis=-1, keepdims=True)
        tbs.append(c)
        key = jnp.where(cols == c, _INT_MIN, key)
    tb = jnp.concatenate(tbs, axis=1)
    tb_ref[...] = tb
    rows = i * _RB + lax.broadcasted_iota(jnp.int32, (_RB, T), 0)
    fl_ref[...] = rows * NBLK + tb


def _selblocks(bm):
    return pl.pallas_call(
        _selblocks_body,
        grid=(B // _RB,),
        in_specs=[pl.BlockSpec((_RB, NBLK), lambda i: (i, 0))],
        out_specs=[
            pl.BlockSpec((_RB, T), lambda i: (i, 0)),
            pl.BlockSpec((_RB, T), lambda i: (i, 0)),
        ],
        out_shape=[
            jax.ShapeDtypeStruct((B, T), jnp.int32),
            jax.ShapeDtypeStruct((B, T), jnp.int32),
        ],
    )(bm)


_NW = 32
_CHUNK3 = B * T // _NW
_IDXROWS3 = _CHUNK3 // 128


def _sc_gather_blocks(scores_flat, flat_idx2):
    mesh = plsc.VectorSubcoreMesh(core_axis_name="c", subcore_axis_name="s")

    @functools.partial(
        pl.kernel, mesh=mesh,
        out_type=jax.ShapeDtypeStruct((B * T, 128), jnp.float32),
        scratch_types=[
            pltpu.VMEM((_IDXROWS3, 128), jnp.int32),
            pltpu.VMEM((_CHUNK3, 128), jnp.float32),
            pltpu.SemaphoreType.DMA,
        ],
    )
    def k(scores_hbm, idx_hbm, out_hbm, idx_v, rows_v, sem):
        wid = lax.axis_index("s") * 2 + lax.axis_index("c")
        pltpu.sync_copy(idx_hbm.at[wid], idx_v)
        for t in range(_IDXROWS3):
            pltpu.async_copy(scores_hbm.at[idx_v.at[t]],
                             rows_v.at[pl.ds(t * 128, 128)], sem).wait()
        pltpu.sync_copy(rows_v, out_hbm.at[pl.ds(wid * _CHUNK3, _CHUNK3)])

    return k(scores_flat, flat_idx2)


def _extract_body(cand_ref, tb_ref, out_ref):
    key = _sortable(cand_ref[...])
    tb3 = tb_ref[...][:, :, None]
    lane = lax.broadcasted_iota(jnp.int32, (_RB, T, 128), 2)
    colidx = tb3 * 128 + lane
    key = jnp.where(colidx < ANUM, key, _INT_MIN)
    outs = []
    for _ in range(K):
        m2 = jnp.max(key, axis=2, keepdims=True)
        m = jnp.max(m2, axis=1, keepdims=True)
        c2 = jnp.min(jnp.where(key == m, colidx, _BIG), axis=2, keepdims=True)
        c = jnp.min(c2, axis=1, keepdims=True)
        outs.append(c[:, 0, :])
        key = jnp.where(colidx == c, _INT_MIN, key)
    out_ref[...] = jnp.concatenate(outs, axis=1)


def _extract(cand, tb):
    return pl.pallas_call(
        _extract_body,
        grid=(B // _RB,),
        in_specs=[
            pl.BlockSpec((_RB, T, 128), lambda i: (i, 0, 0)),
            pl.BlockSpec((_RB, T), lambda i: (i, 0)),
        ],
        out_specs=pl.BlockSpec((_RB, K), lambda i: (i, 0)),
        out_shape=jax.ShapeDtypeStruct((B, K), jnp.int32),
    )(cand, tb)


_CHUNK5 = B * K // _NW
_IDXROWS5 = _CHUNK5 // 128


def _sc_gather_emb(data_emb, idx2):
    mesh = plsc.VectorSubcoreMesh(core_axis_name="c", subcore_axis_name="s")

    @functools.partial(
        pl.kernel, mesh=mesh,
        out_type=jax.ShapeDtypeStruct((B * K, DM), jnp.float32),
        scratch_types=[
            pltpu.VMEM((_IDXROWS5, 128), jnp.int32),
            pltpu.VMEM((_CHUNK5, DM), jnp.float32),
            pltpu.SemaphoreType.DMA,
        ],
    )
    def k(table_hbm, idx_hbm, out_hbm, idx_v, rows_v, sem):
        wid = lax.axis_index("s") * 2 + lax.axis_index("c")
        pltpu.sync_copy(idx_hbm.at[wid], idx_v)
        for t in range(_IDXROWS5):
            pltpu.async_copy(table_hbm.at[idx_v.at[t]],
                             rows_v.at[pl.ds(t * 128, 128)], sem).wait()
        pltpu.sync_copy(rows_v, out_hbm.at[pl.ds(wid * _CHUNK5, _CHUNK5)])

    return k(data_emb, idx2)


def _pooling(x, pooler_w):
    length_mask = jnp.where(jnp.sum(x, axis=-1) == 0, 0, 1)
    weight = x @ pooler_w.T
    weight = jnp.where(length_mask[..., None] == 1, weight, -1000000000.0)
    return jnp.sum(jax.nn.softmax(weight, axis=1) * x, axis=1)


def _pad_items(item_emb):
    n = item_emb.shape[0]
    item_pad = jnp.pad(item_emb, ((0, NPAD - n), (0, 0)))
    ni = jnp.linalg.norm(item_emb, axis=-1)
    ni_pad = jnp.broadcast_to(
        jnp.pad(ni, (0, NPAD - n), constant_values=1.0)[None, :], (8, NPAD))
    return item_pad, ni_pad


def kernel(gen_embedding, domains, proj_A, proj_B, pooler_A, pooler_B, A_item_emb, B_item_emb, data_emb):
    A_gen = _pooling(gen_embedding @ proj_A.T, pooler_A)
    B_gen = _pooling(gen_embedding @ proj_B.T, pooler_B)

    dom0 = domains == 0
    perm = jnp.argsort(jnp.where(dom0, 0, 1), stable=True)
    inv = jnp.argsort(perm, stable=True)
    cnt0 = jnp.sum(dom0).astype(jnp.int32)
    nb0 = (cnt0 + _RB - 1) // _RB
    ii = jnp.arange(B // _RB, dtype=jnp.int32)
    tsel = jnp.where(ii < nb0 - 1, 0, jnp.where(ii == nb0 - 1, 2, 1))
    cnt0arr = cnt0[None]

    gen_sel = jnp.where(dom0[:, None], B_gen, A_gen)
    gperm = jnp.take(gen_sel, perm, axis=0)
    ngA = jnp.linalg.norm(A_gen, axis=-1, keepdims=True)
    ngB = jnp.linalg.norm(B_gen, axis=-1, keepdims=True)
    ng_sel = jnp.where(dom0[:, None], ngB, ngA)
    ngperm = jnp.take(ng_sel, perm, axis=0)

    itemA_pad, niA_pad = _pad_items(A_item_emb)
    itemB_pad, niB_pad = _pad_items(B_item_emb)

    scores3, bm4 = _simscore(gperm, itemA_pad, itemB_pad, ngperm,
                             niA_pad, niB_pad, tsel, cnt0arr)
    tb, fl = _selblocks(bm4.reshape(B, NBLK))
    cand = _sc_gather_blocks(scores3.reshape(B * NBLK, 128),
                             fl.reshape(_NW, _IDXROWS3, 128))
    idx_perm = _extract(cand.reshape(B, T, 128), tb)
    idx_nat = jnp.take(idx_perm, inv, axis=0)

    recalled = jnp.where(domains[:, None] == 0, idx_nat + 1 + ANUM, idx_nat + 1)
    emb = _sc_gather_emb(data_emb, recalled.reshape(_NW, _IDXROWS5, 128))
    recalled_emb = emb.reshape(B, K, DM)
    return recalled, recalled_emb
